# baseline (speedup 1.0000x reference)
import jax
import jax.numpy as jnp
from jax import lax
from jax.experimental import pallas as pl
from jax.experimental.pallas import tpu as pltpu
from jax.experimental.pallas import tpu_sc as plsc

EMB = 100
EMB_PAD = 112
PAD = 128
NUM_CORES = 2
NUM_SUBCORES = 16
NW = NUM_CORES * NUM_SUBCORES
LANES = 128


def _pad_body(t_ref, o_ref):
    o_ref[:, :EMB] = t_ref[...].T


def _pad_table(table_t):
    v = table_t.shape[1]
    blk = 16384
    return pl.pallas_call(
        _pad_body,
        grid=(pl.cdiv(v, blk),),
        in_specs=[pl.BlockSpec((EMB, blk), lambda i: (0, i))],
        out_specs=pl.BlockSpec((blk, PAD), lambda i: (i, 0)),
        out_shape=jax.ShapeDtypeStruct((v, PAD), jnp.float32),
    )(table_t)


def _gather_body(table_hbm, xt_hbm, out_hbm, idx_v, pad_v, trans_v, sems):
    hist = xt_hbm.shape[0]
    wid = lax.axis_index("s") * NUM_CORES + lax.axis_index("c")
    b0 = wid * LANES
    sem_g0, sem_g1, sem_g2, sem_g3, sem_w0, sem_w1 = sems
    pltpu.sync_copy(xt_hbm.at[:, pl.ds(b0, LANES)], idx_v)

    lane = lax.iota(jnp.int32, 16)
    perms = [(lane + s) & 15 for s in range(16)]

    def transpose(src, dst):
        nbt = LANES // 16

        def blk_body(m, c):
            et = m >> 3
            bt = m & 7
            e0 = et * 16
            sblk = src.at[pl.ds(bt * 16, 16)]
            dblk = dst.at[pl.ds(e0, 16)]
            rowv = lane + bt * 16
            pend = []
            for s in range(16):
                pend.append((s, plsc.load_gather(sblk, [lane, e0 + perms[s]])))
                if len(pend) > 4:
                    ps, pv = pend.pop(0)
                    plsc.store_scatter(dblk, [perms[ps], rowv], pv)
            for ps, pv in pend:
                plsc.store_scatter(dblk, [perms[ps], rowv], pv)
            return c
        lax.fori_loop(0, (EMB_PAD // 16) * nbt, blk_body, 0)

    def gath(h, buf, sem):
        return pltpu.make_async_copy(
            table_hbm.at[idx_v.at[h]], pad_v.at[buf], sem)

    def wr(h, buf, sem):
        return pltpu.make_async_copy(
            trans_v.at[buf, pl.ds(0, EMB)],
            out_hbm.at[:, h, pl.ds(b0, LANES)], sem)

    n4 = hist // 4
    gsems = (sem_g0, sem_g1, sem_g2, sem_g3)
    gath(0, 0, gsems[0]).start()
    gath(1, 1, gsems[1]).start()

    def process(h, pb, first):
        gath(h, pb, gsems[pb]).wait()
        pl.when(h >= 2)(lambda: wr(h - 2, 0 if first else 1,
                                   sem_w0 if first else sem_w1).wait())
        transpose(pad_v.at[pb], trans_v.at[0 if first else 1])
        wr(h, 0 if first else 1, sem_w0 if first else sem_w1).start()

    def body(q, carry):
        h0 = q * 4
        gath(h0 + 2, 2, gsems[2]).start()
        gath(h0 + 3, 3, gsems[3]).start()
        process(h0, 0, True)
        process(h0 + 1, 1, False)

        def prefetch():
            gath(h0 + 4, 0, gsems[0]).start()
            gath(h0 + 5, 1, gsems[1]).start()
        pl.when(q + 1 < n4)(prefetch)
        process(h0 + 2, 2, True)
        process(h0 + 3, 3, False)
        return carry

    lax.fori_loop(0, n4, body, 0)
    wr(hist - 2, 0, sem_w0).wait()
    wr(hist - 1, 1, sem_w1).wait()


@jax.jit
def _run(table_t, xt):
    hist = xt.shape[0]
    mesh = plsc.VectorSubcoreMesh(core_axis_name="c", subcore_axis_name="s")
    table_padded = _pad_table(table_t)
    f = pl.kernel(
        _gather_body,
        mesh=mesh,
        compiler_params=pltpu.CompilerParams(needs_layout_passes=False),
        out_type=jax.ShapeDtypeStruct((EMB, hist, NW * LANES), jnp.float32),
        scratch_types=[
            pltpu.VMEM((hist, LANES), jnp.int32),
            pltpu.VMEM((4, LANES, PAD), jnp.float32),
            pltpu.VMEM((2, EMB_PAD, LANES), jnp.float32),
            [pltpu.SemaphoreType.DMA] * 6,
        ],
    )
    return f(table_padded, xt)


def kernel(table, x):
    b, h = x.shape
    assert b == NW * LANES and h % 4 == 0
    out_t = _run(table.T, x.astype(jnp.int32).T)
    return out_t.transpose(2, 1, 0)


# speedup vs baseline: 1.0054x; 1.0054x over previous
import jax
import jax.numpy as jnp
from jax import lax
from jax.experimental import pallas as pl
from jax.experimental.pallas import tpu as pltpu
from jax.experimental.pallas import tpu_sc as plsc

EMB = 100
EMB_PAD = 112
PAD = 128
NUM_CORES = 2
NUM_SUBCORES = 16
NW = NUM_CORES * NUM_SUBCORES
LANES = 128


def _pad_body(t_ref, o_ref):
    o_ref[:, :EMB] = t_ref[...].T


def _pad_table(table_t):
    v = table_t.shape[1]
    blk = 32768
    return pl.pallas_call(
        _pad_body,
        grid=(pl.cdiv(v, blk),),
        in_specs=[pl.BlockSpec((EMB, blk), lambda i: (0, i))],
        out_specs=pl.BlockSpec((blk, PAD), lambda i: (i, 0)),
        out_shape=jax.ShapeDtypeStruct((v, PAD), jnp.float32),
    )(table_t)


def _gather_body(table_hbm, xt_hbm, out_hbm, idx_v, pad_v, trans_v, sems):
    hist = xt_hbm.shape[0]
    wid = lax.axis_index("s") * NUM_CORES + lax.axis_index("c")
    b0 = wid * LANES
    sem_g0, sem_g1, sem_g2, sem_g3, sem_w0, sem_w1 = sems
    pltpu.sync_copy(xt_hbm.at[:, pl.ds(b0, LANES)], idx_v)

    lane = lax.iota(jnp.int32, 16)
    perms = [(lane + s) & 15 for s in range(16)]

    def transpose(src, dst):
        nbt = LANES // 16

        def blk_body(m, c):
            et = m >> 3
            bt = m & 7
            e0 = et * 16
            sblk = src.at[pl.ds(bt * 16, 16)]
            dblk = dst.at[pl.ds(e0, 16)]
            rowv = lane + bt * 16
            pend = []
            for s in range(16):
                pend.append((s, plsc.load_gather(sblk, [lane, e0 + perms[s]])))
                if len(pend) > 4:
                    ps, pv = pend.pop(0)
                    plsc.store_scatter(dblk, [perms[ps], rowv], pv)
            for ps, pv in pend:
                plsc.store_scatter(dblk, [perms[ps], rowv], pv)
            return c
        lax.fori_loop(0, (EMB_PAD // 16) * nbt, blk_body, 0)

    def gath(h, buf, sem):
        return pltpu.make_async_copy(
            table_hbm.at[idx_v.at[h]], pad_v.at[buf], sem)

    def wr(h, buf, sem):
        return pltpu.make_async_copy(
            trans_v.at[buf, pl.ds(0, EMB)],
            out_hbm.at[:, h, pl.ds(b0, LANES)], sem)

    n4 = hist // 4
    gsems = (sem_g0, sem_g1, sem_g2, sem_g3)
    gath(0, 0, gsems[0]).start()
    gath(1, 1, gsems[1]).start()

    def process(h, pb, first):
        gath(h, pb, gsems[pb]).wait()
        pl.when(h >= 2)(lambda: wr(h - 2, 0 if first else 1,
                                   sem_w0 if first else sem_w1).wait())
        transpose(pad_v.at[pb], trans_v.at[0 if first else 1])
        wr(h, 0 if first else 1, sem_w0 if first else sem_w1).start()

    def body(q, carry):
        h0 = q * 4
        gath(h0 + 2, 2, gsems[2]).start()
        gath(h0 + 3, 3, gsems[3]).start()
        process(h0, 0, True)
        process(h0 + 1, 1, False)

        def prefetch():
            gath(h0 + 4, 0, gsems[0]).start()
            gath(h0 + 5, 1, gsems[1]).start()
        pl.when(q + 1 < n4)(prefetch)
        process(h0 + 2, 2, True)
        process(h0 + 3, 3, False)
        return carry

    lax.fori_loop(0, n4, body, 0)
    wr(hist - 2, 0, sem_w0).wait()
    wr(hist - 1, 1, sem_w1).wait()


@jax.jit
def _run(table_t, xt):
    hist = xt.shape[0]
    mesh = plsc.VectorSubcoreMesh(core_axis_name="c", subcore_axis_name="s")
    table_padded = _pad_table(table_t)
    f = pl.kernel(
        _gather_body,
        mesh=mesh,
        compiler_params=pltpu.CompilerParams(needs_layout_passes=False),
        out_type=jax.ShapeDtypeStruct((EMB, hist, NW * LANES), jnp.float32),
        scratch_types=[
            pltpu.VMEM((hist, LANES), jnp.int32),
            pltpu.VMEM((4, LANES, PAD), jnp.float32),
            pltpu.VMEM((2, EMB_PAD, LANES), jnp.float32),
            [pltpu.SemaphoreType.DMA] * 6,
        ],
    )
    return f(table_padded, xt)


def kernel(table, x):
    b, h = x.shape
    assert b == NW * LANES and h % 4 == 0
    out_t = _run(table.T, x.astype(jnp.int32).T)
    return out_t.transpose(2, 1, 0)
